# Initial kernel scaffold; baseline (speedup 1.0000x reference)
import functools

import jax
import jax.numpy as jnp
from jax import lax
from jax.experimental import pallas as pl
from jax.experimental.pallas import tpu as pltpu
from jax.experimental.pallas import tpu_sc as plsc

_N = 10000
_E = 320000
_H = 128
_L = 4

_NC = 2
_NS = 16
_NW = _NC * _NS
_C = 128
_NFULL = 78
_EW = _NFULL * _C
_TAIL0 = _NW * _EW
_NTAIL = (_E - _TAIL0) // _C


def _sc_mesh():
    return plsc.VectorSubcoreMesh(
        core_axis_name="c", subcore_axis_name="s",
        num_cores=_NC, num_subcores=_NS)


def _make_gather(width):
    @functools.partial(
        pl.kernel,
        out_type=jax.ShapeDtypeStruct((_E, width), jnp.float32),
        mesh=_sc_mesh(),
        scratch_types=[
            pltpu.VMEM((_EW,), jnp.int32),
            pltpu.VMEM((_C,), jnp.int32),
            pltpu.VMEM((_C, width), jnp.float32),
            pltpu.SemaphoreType.DMA,
        ],
    )
    def gather(tbl, idx_hbm, out, idx_all, idx_t, buf, sem):
        w = lax.axis_index("s") * _NC + lax.axis_index("c")
        base = pl.multiple_of(w * _EW, _EW)
        pltpu.sync_copy(idx_hbm.at[pl.ds(base, _EW)], idx_all)

        def step(k, carry):
            off = pl.multiple_of(k * _C, _C)
            pltpu.async_copy(tbl.at[idx_all.at[pl.ds(off, _C)]], buf, sem).wait()
            pltpu.sync_copy(buf, out.at[pl.ds(base + off, _C)])
            return carry

        lax.fori_loop(0, _NFULL, step, 0)

        @pl.when(w < _NTAIL)
        def _tail():
            tb = pl.multiple_of(_TAIL0 + w * _C, _C)
            pltpu.sync_copy(idx_hbm.at[pl.ds(tb, _C)], idx_t)
            pltpu.async_copy(tbl.at[idx_t], buf, sem).wait()
            pltpu.sync_copy(buf, out.at[pl.ds(tb, _C)])

    return gather


def _make_scatter(width, nzrow):
    nz = _N // nzrow

    @functools.partial(
        pl.kernel,
        out_type=jax.ShapeDtypeStruct((_NC * _N, width), jnp.float32),
        mesh=_sc_mesh(),
        scratch_types=[
            pltpu.VMEM_SHARED((_N, width), jnp.float32),
            pltpu.VMEM((_EW,), jnp.int32),
            pltpu.VMEM((_C,), jnp.int32),
            pltpu.VMEM((_C, width), jnp.float32),
            pltpu.SemaphoreType.DMA,
        ],
    )
    def scatter(msg, idx_hbm, zeros_hbm, out, acc, idx_all, idx_t, buf, sem):
        cid = lax.axis_index("c")
        sid = lax.axis_index("s")
        w = sid * _NC + cid

        @pl.when(sid < nz)
        def _zero():
            pltpu.sync_copy(zeros_hbm, acc.at[pl.ds(sid * nzrow, nzrow)])

        plsc.subcore_barrier()

        base = pl.multiple_of(w * _EW, _EW)
        pltpu.sync_copy(idx_hbm.at[pl.ds(base, _EW)], idx_all)

        def step(k, carry):
            off = pl.multiple_of(k * _C, _C)
            pltpu.sync_copy(msg.at[pl.ds(base + off, _C)], buf)
            pltpu.sync_copy(buf, acc.at[idx_all.at[pl.ds(off, _C)]], add=True)
            return carry

        lax.fori_loop(0, _NFULL, step, 0)

        @pl.when(w < _NTAIL)
        def _tail():
            tb = pl.multiple_of(_TAIL0 + w * _C, _C)
            pltpu.sync_copy(idx_hbm.at[pl.ds(tb, _C)], idx_t)
            pltpu.sync_copy(msg.at[pl.ds(tb, _C)], buf)
            pltpu.sync_copy(buf, acc.at[idx_t], add=True)

        plsc.subcore_barrier()

        @pl.when(sid < nz)
        def _writeback():
            pltpu.sync_copy(acc.at[pl.ds(sid * nzrow, nzrow)],
                            out.at[pl.ds(cid * _N + sid * nzrow, nzrow)])

    return scatter


def _make_scatter2():
    width = 8
    nzrow = 1250
    nz = _N // nzrow

    @functools.partial(
        pl.kernel,
        out_type=jax.ShapeDtypeStruct((_NC * _N, width), jnp.float32),
        mesh=_sc_mesh(),
        scratch_types=[
            pltpu.VMEM_SHARED((_N, width), jnp.float32),
            pltpu.VMEM((_EW,), jnp.int32),
            pltpu.VMEM((_EW,), jnp.int32),
            pltpu.VMEM((_C,), jnp.int32),
            pltpu.VMEM((_C, width), jnp.float32),
            pltpu.SemaphoreType.DMA,
        ],
    )
    def scatter2(pf, nf, row_hbm, col_hbm, zeros_hbm, out,
                 acc, ridx_all, cidx_all, idx_t, buf, sem):
        cid = lax.axis_index("c")
        sid = lax.axis_index("s")
        w = sid * _NC + cid

        @pl.when(sid < nz)
        def _zero():
            pltpu.sync_copy(zeros_hbm, acc.at[pl.ds(sid * nzrow, nzrow)])

        plsc.subcore_barrier()

        base = pl.multiple_of(w * _EW, _EW)
        pltpu.sync_copy(row_hbm.at[pl.ds(base, _EW)], ridx_all)
        pltpu.sync_copy(col_hbm.at[pl.ds(base, _EW)], cidx_all)

        def step(k, carry):
            off = pl.multiple_of(k * _C, _C)
            pltpu.sync_copy(pf.at[pl.ds(base + off, _C)], buf)
            pltpu.sync_copy(buf, acc.at[ridx_all.at[pl.ds(off, _C)]], add=True)
            pltpu.sync_copy(nf.at[pl.ds(base + off, _C)], buf)
            pltpu.sync_copy(buf, acc.at[cidx_all.at[pl.ds(off, _C)]], add=True)
            return carry

        lax.fori_loop(0, _NFULL, step, 0)

        @pl.when(w < _NTAIL)
        def _tail():
            tb = pl.multiple_of(_TAIL0 + w * _C, _C)
            pltpu.sync_copy(pf.at[pl.ds(tb, _C)], buf)
            pltpu.sync_copy(row_hbm.at[pl.ds(tb, _C)], idx_t)
            pltpu.sync_copy(buf, acc.at[idx_t], add=True)
            pltpu.sync_copy(nf.at[pl.ds(tb, _C)], buf)
            pltpu.sync_copy(col_hbm.at[pl.ds(tb, _C)], idx_t)
            pltpu.sync_copy(buf, acc.at[idx_t], add=True)

        plsc.subcore_barrier()

        @pl.when(sid < nz)
        def _writeback():
            pltpu.sync_copy(acc.at[pl.ds(sid * nzrow, nzrow)],
                            out.at[pl.ds(cid * _N + sid * nzrow, nzrow)])

    return scatter2


_gather8 = _make_gather(8)
_gather128 = _make_gather(_H)
_scatter128 = _make_scatter(_H, _N // _NS)
_scatter2 = _make_scatter2()


_BN = 1000
_GN = _N // _BN
_BE = 2000
_GE = _E // _BE

_PAR = pltpu.CompilerParams(dimension_semantics=("parallel",))


def _silu(x):
    return x * jax.nn.sigmoid(x)


def _full(shape):
    return pl.BlockSpec(shape, lambda i: (0, 0))


def _g0_body(vel_ref, w_ref, b_ref, out_ref):
    v = vel_ref[...]
    s = jnp.sum(w_ref[...], axis=0, keepdims=True)
    vn = jnp.sqrt(jnp.sum(v * v, axis=1, keepdims=True))
    out_ref[...] = vn * s + b_ref[...]


_g0_call = pl.pallas_call(
    _g0_body,
    grid=(_GN,),
    in_specs=[pl.BlockSpec((_BN, 3), lambda i: (i, 0)),
              _full((_H, _H)), _full((1, _H))],
    out_specs=pl.BlockSpec((_BN, _H), lambda i: (i, 0)),
    out_shape=jax.ShapeDtypeStruct((_N, _H), jnp.float32),
    compiler_params=_PAR,
)


def _g_body(p0_ref, p1_ref, w_ref, b_ref, out_ref):
    nf = p0_ref[...] + p1_ref[...]
    out_ref[...] = (jnp.dot(nf, w_ref[...], preferred_element_type=jnp.float32)
                    + b_ref[...])


_g_call = pl.pallas_call(
    _g_body,
    grid=(_GN,),
    in_specs=[pl.BlockSpec((_BN, _H), lambda i: (i, 0)),
              pl.BlockSpec((_BN, _H), lambda i: (i, 0)),
              _full((_H, _H)), _full((1, _H))],
    out_specs=pl.BlockSpec((_BN, _H), lambda i: (i, 0)),
    out_shape=jax.ShapeDtypeStruct((_N, _H), jnp.float32),
    compiler_params=_PAR,
)


def _edge_attr(pr, pc):
    rd = pr - pc
    d = jnp.sqrt(jnp.sum(rd * rd, axis=1, keepdims=True))
    return rd, d


def _am0_body(posr_ref, posc_ref, g_ref, eew1_ref, eeb1_ref, eew2_ref,
              eeb2_ref, w1t_ref, w2_ref, b2_ref, emb_ref, msg_ref):
    rd, d = _edge_attr(posr_ref[...], posc_ref[...])
    lane = lax.broadcasted_iota(jnp.int32, rd.shape, 1)
    ea = jnp.where(lane < 3, rd, jnp.where(lane == 3, d, 0.0))
    h = _silu(jnp.dot(ea, eew1_ref[...], preferred_element_type=jnp.float32)
              + eeb1_ref[...])
    emb = (jnp.dot(h, eew2_ref[...], preferred_element_type=jnp.float32)
           + eeb2_ref[...])
    emb_ref[...] = emb
    h2 = _silu(jnp.dot(emb, w1t_ref[...], preferred_element_type=jnp.float32)
               + g_ref[...])
    msg_ref[...] = (jnp.dot(h2, w2_ref[...], preferred_element_type=jnp.float32)
                    + b2_ref[...])


_am0_call = pl.pallas_call(
    _am0_body,
    grid=(_GE,),
    in_specs=[pl.BlockSpec((_BE, 8), lambda i: (i, 0)),
              pl.BlockSpec((_BE, 8), lambda i: (i, 0)),
              pl.BlockSpec((_BE, _H), lambda i: (i, 0)),
              _full((8, _H)), _full((1, _H)), _full((_H, _H)), _full((1, _H)),
              _full((_H, _H)), _full((_H, _H)), _full((1, _H))],
    out_specs=[pl.BlockSpec((_BE, _H), lambda i: (i, 0)),
               pl.BlockSpec((_BE, _H), lambda i: (i, 0))],
    out_shape=[jax.ShapeDtypeStruct((_E, _H), jnp.float32),
               jax.ShapeDtypeStruct((_E, _H), jnp.float32)],
    compiler_params=_PAR,
)


def _msg_body(emb_ref, g_ref, w1t_ref, w2_ref, b2_ref, msg_ref):
    h = _silu(jnp.dot(emb_ref[...], w1t_ref[...],
                      preferred_element_type=jnp.float32) + g_ref[...])
    msg_ref[...] = (jnp.dot(h, w2_ref[...], preferred_element_type=jnp.float32)
                    + b2_ref[...])


_msg_call = pl.pallas_call(
    _msg_body,
    grid=(_GE,),
    in_specs=[pl.BlockSpec((_BE, _H), lambda i: (i, 0)),
              pl.BlockSpec((_BE, _H), lambda i: (i, 0)),
              _full((_H, _H)), _full((_H, _H)), _full((1, _H))],
    out_specs=pl.BlockSpec((_BE, _H), lambda i: (i, 0)),
    out_shape=jax.ShapeDtypeStruct((_E, _H), jnp.float32),
    compiler_params=_PAR,
)


def _ff_body(g_ref, w2_ref, b2_ref, posr_ref, posc_ref, pf_ref, nf_ref):
    fm8 = (jnp.dot(_silu(g_ref[...]), w2_ref[...],
                   preferred_element_type=jnp.float32) + b2_ref[...])
    fm = fm8[:, 0:1]
    rd, d = _edge_attr(posr_ref[...], posc_ref[...])
    pf = fm * (rd / (d + 1e-8))
    pf_ref[...] = pf
    nf_ref[...] = -pf


_ff_call = pl.pallas_call(
    _ff_body,
    grid=(_GE,),
    in_specs=[pl.BlockSpec((_BE, _H), lambda i: (i, 0)),
              _full((_H, 8)), _full((1, 8)),
              pl.BlockSpec((_BE, 8), lambda i: (i, 0)),
              pl.BlockSpec((_BE, 8), lambda i: (i, 0))],
    out_specs=[pl.BlockSpec((_BE, 8), lambda i: (i, 0)),
               pl.BlockSpec((_BE, 8), lambda i: (i, 0))],
    out_shape=[jax.ShapeDtypeStruct((_E, 8), jnp.float32),
               jax.ShapeDtypeStruct((_E, 8), jnp.float32)],
    compiler_params=_PAR,
)


def _fin_body(q0_ref, q1_ref, out_ref):
    out_ref[...] = (q0_ref[...] + q1_ref[...])[:, :3]


_fin_call = pl.pallas_call(
    _fin_body,
    grid=(_GN,),
    in_specs=[pl.BlockSpec((_BN, 8), lambda i: (i, 0)),
              pl.BlockSpec((_BN, 8), lambda i: (i, 0))],
    out_specs=pl.BlockSpec((_BN, 3), lambda i: (i, 0)),
    out_shape=jax.ShapeDtypeStruct((_N, 3), jnp.float32),
    compiler_params=_PAR,
)


def kernel(pos, vel, masses, edge_index, ee_w1, ee_b1, ee_w2, ee_b2,
           msg_w1, msg_b1, msg_w2, msg_b2, fd_w1, fd_b1, fd_w2, fd_b2):
    f32 = jnp.float32
    row = edge_index[0]
    col = edge_index[1]
    pos8 = jnp.concatenate([pos, jnp.zeros((_N, 5), f32)], axis=1)
    eew1p = jnp.concatenate([ee_w1, jnp.zeros((4, _H), f32)], axis=0)
    fd_w2p = jnp.concatenate([fd_w2, jnp.zeros((_H, 7), f32)], axis=1)
    fd_b2p = jnp.concatenate([fd_b2, jnp.zeros((7,), f32)]).reshape(1, 8)
    w1t = msg_w1[:, :_H, :]
    w1b = msg_w1[:, _H:, :]
    zeros128 = jnp.zeros((_N // _NS, _H), f32)
    zeros8 = jnp.zeros((1250, 8), f32)

    g0 = _g0_call(vel, w1b[0], msg_b1[0].reshape(1, _H))
    posr = _gather8(pos8, row)
    posc = _gather8(pos8, col)
    gth = _gather128(g0, row)
    emb, msg = _am0_call(posr, posc, gth, eew1p, ee_b1.reshape(1, _H),
                         ee_w2, ee_b2.reshape(1, _H), w1t[0], msg_w2[0],
                         msg_b2[0].reshape(1, _H))
    p = _scatter128(msg, col, zeros128)
    for l in range(1, _L):
        g = _g_call(p[:_N], p[_N:], w1b[l], msg_b1[l].reshape(1, _H))
        gth = _gather128(g, row)
        msg = _msg_call(emb, gth, w1t[l], msg_w2[l], msg_b2[l].reshape(1, _H))
        p = _scatter128(msg, col, zeros128)
    gf = _g_call(p[:_N], p[_N:], fd_w1, fd_b1.reshape(1, _H))
    gfr = _gather128(gf, row)
    pf, nf = _ff_call(gfr, fd_w2p, fd_b2p, posr, posc)
    q = _scatter2(pf, nf, row, col, zeros8)
    return _fin_call(q[:_N], q[_N:])


# speedup vs baseline: 2.8083x; 2.8083x over previous
import functools

import jax
import jax.numpy as jnp
from jax import lax
from jax.experimental import pallas as pl
from jax.experimental.pallas import tpu as pltpu
from jax.experimental.pallas import tpu_sc as plsc

_N = 10000
_E = 320000
_H = 128
_L = 4

_NC = 2
_NS = 16
_NW = _NC * _NS
_C = 128
_NFULL = 78
_EW = _NFULL * _C
_TAIL0 = _NW * _EW
_NTAIL = (_E - _TAIL0) // _C


def _sc_mesh():
    return plsc.VectorSubcoreMesh(
        core_axis_name="c", subcore_axis_name="s",
        num_cores=_NC, num_subcores=_NS)


_SC_PARAMS = pltpu.CompilerParams(use_tc_tiling_on_sc=False)


def _make_gather(width):
    @functools.partial(
        pl.kernel,
        out_type=jax.ShapeDtypeStruct((_E, width), jnp.float32),
        mesh=_sc_mesh(),
        compiler_params=_SC_PARAMS,
        scratch_types=[
            pltpu.VMEM((_EW,), jnp.int32),
            pltpu.VMEM((_C,), jnp.int32),
            pltpu.VMEM((_C, width), jnp.float32),
            pltpu.SemaphoreType.DMA,
        ],
    )
    def gather(tbl, idx_hbm, out, idx_all, idx_t, buf, sem):
        w = lax.axis_index("s") * _NC + lax.axis_index("c")
        base = pl.multiple_of(w * _EW, _EW)
        pltpu.sync_copy(idx_hbm.at[pl.ds(base, _EW)], idx_all)

        def step(k, carry):
            off = pl.multiple_of(k * _C, _C)
            pltpu.async_copy(tbl.at[idx_all.at[pl.ds(off, _C)]], buf, sem).wait()
            pltpu.sync_copy(buf, out.at[pl.ds(base + off, _C)])
            return carry

        lax.fori_loop(0, _NFULL, step, 0)

        @pl.when(w < _NTAIL)
        def _tail():
            tb = pl.multiple_of(_TAIL0 + w * _C, _C)
            pltpu.sync_copy(idx_hbm.at[pl.ds(tb, _C)], idx_t)
            pltpu.async_copy(tbl.at[idx_t], buf, sem).wait()
            pltpu.sync_copy(buf, out.at[pl.ds(tb, _C)])

    return gather


def _make_scatter(width, nzrow):
    nz = _N // nzrow

    @functools.partial(
        pl.kernel,
        out_type=jax.ShapeDtypeStruct((_NC * _N, width), jnp.float32),
        mesh=_sc_mesh(),
        compiler_params=_SC_PARAMS,
        scratch_types=[
            pltpu.VMEM_SHARED((_N, width), jnp.float32),
            pltpu.VMEM((_EW,), jnp.int32),
            pltpu.VMEM((_C,), jnp.int32),
            pltpu.VMEM((_C, width), jnp.float32),
            pltpu.SemaphoreType.DMA,
        ],
    )
    def scatter(msg, idx_hbm, zeros_hbm, out, acc, idx_all, idx_t, buf, sem):
        cid = lax.axis_index("c")
        sid = lax.axis_index("s")
        w = sid * _NC + cid

        @pl.when(sid < nz)
        def _zero():
            pltpu.sync_copy(zeros_hbm, acc.at[pl.ds(sid * nzrow, nzrow)])

        plsc.subcore_barrier()

        base = pl.multiple_of(w * _EW, _EW)
        pltpu.sync_copy(idx_hbm.at[pl.ds(base, _EW)], idx_all)

        def step(k, carry):
            off = pl.multiple_of(k * _C, _C)
            pltpu.sync_copy(msg.at[pl.ds(base + off, _C)], buf)
            pltpu.sync_copy(buf, acc.at[idx_all.at[pl.ds(off, _C)]], add=True)
            return carry

        lax.fori_loop(0, _NFULL, step, 0)

        @pl.when(w < _NTAIL)
        def _tail():
            tb = pl.multiple_of(_TAIL0 + w * _C, _C)
            pltpu.sync_copy(idx_hbm.at[pl.ds(tb, _C)], idx_t)
            pltpu.sync_copy(msg.at[pl.ds(tb, _C)], buf)
            pltpu.sync_copy(buf, acc.at[idx_t], add=True)

        plsc.subcore_barrier()

        @pl.when(sid < nz)
        def _writeback():
            pltpu.sync_copy(acc.at[pl.ds(sid * nzrow, nzrow)],
                            out.at[pl.ds(cid * _N + sid * nzrow, nzrow)])

    return scatter


def _make_scatter2():
    width = 8
    nzrow = 1250
    nz = _N // nzrow

    @functools.partial(
        pl.kernel,
        out_type=jax.ShapeDtypeStruct((_NC * _N, width), jnp.float32),
        mesh=_sc_mesh(),
        compiler_params=_SC_PARAMS,
        scratch_types=[
            pltpu.VMEM_SHARED((_N, width), jnp.float32),
            pltpu.VMEM((_EW,), jnp.int32),
            pltpu.VMEM((_EW,), jnp.int32),
            pltpu.VMEM((_C,), jnp.int32),
            pltpu.VMEM((_C, width), jnp.float32),
            pltpu.SemaphoreType.DMA,
        ],
    )
    def scatter2(pf, nf, row_hbm, col_hbm, zeros_hbm, out,
                 acc, ridx_all, cidx_all, idx_t, buf, sem):
        cid = lax.axis_index("c")
        sid = lax.axis_index("s")
        w = sid * _NC + cid

        @pl.when(sid < nz)
        def _zero():
            pltpu.sync_copy(zeros_hbm, acc.at[pl.ds(sid * nzrow, nzrow)])

        plsc.subcore_barrier()

        base = pl.multiple_of(w * _EW, _EW)
        pltpu.sync_copy(row_hbm.at[pl.ds(base, _EW)], ridx_all)
        pltpu.sync_copy(col_hbm.at[pl.ds(base, _EW)], cidx_all)

        def step(k, carry):
            off = pl.multiple_of(k * _C, _C)
            pltpu.sync_copy(pf.at[pl.ds(base + off, _C)], buf)
            pltpu.sync_copy(buf, acc.at[ridx_all.at[pl.ds(off, _C)]], add=True)
            pltpu.sync_copy(nf.at[pl.ds(base + off, _C)], buf)
            pltpu.sync_copy(buf, acc.at[cidx_all.at[pl.ds(off, _C)]], add=True)
            return carry

        lax.fori_loop(0, _NFULL, step, 0)

        @pl.when(w < _NTAIL)
        def _tail():
            tb = pl.multiple_of(_TAIL0 + w * _C, _C)
            pltpu.sync_copy(pf.at[pl.ds(tb, _C)], buf)
            pltpu.sync_copy(row_hbm.at[pl.ds(tb, _C)], idx_t)
            pltpu.sync_copy(buf, acc.at[idx_t], add=True)
            pltpu.sync_copy(nf.at[pl.ds(tb, _C)], buf)
            pltpu.sync_copy(col_hbm.at[pl.ds(tb, _C)], idx_t)
            pltpu.sync_copy(buf, acc.at[idx_t], add=True)

        plsc.subcore_barrier()

        @pl.when(sid < nz)
        def _writeback():
            pltpu.sync_copy(acc.at[pl.ds(sid * nzrow, nzrow)],
                            out.at[pl.ds(cid * _N + sid * nzrow, nzrow)])

    return scatter2


_gather8 = _make_gather(8)
_gather128 = _make_gather(_H)
_scatter128 = _make_scatter(_H, _N // _NS)
_scatter2 = _make_scatter2()


_BN = 1000
_GN = _N // _BN
_BE = 2000
_GE = _E // _BE

_PAR = pltpu.CompilerParams(dimension_semantics=("parallel",))


def _silu(x):
    return x * jax.nn.sigmoid(x)


def _full(shape):
    return pl.BlockSpec(shape, lambda i: (0, 0))


def _g0_body(vel_ref, w_ref, b_ref, out_ref):
    v = vel_ref[...]
    s = jnp.sum(w_ref[...], axis=0, keepdims=True)
    vn = jnp.sqrt(jnp.sum(v * v, axis=1, keepdims=True))
    out_ref[...] = vn * s + b_ref[...]


_g0_call = pl.pallas_call(
    _g0_body,
    grid=(_GN,),
    in_specs=[pl.BlockSpec((_BN, 3), lambda i: (i, 0)),
              _full((_H, _H)), _full((1, _H))],
    out_specs=pl.BlockSpec((_BN, _H), lambda i: (i, 0)),
    out_shape=jax.ShapeDtypeStruct((_N, _H), jnp.float32),
    compiler_params=_PAR,
)


def _g_body(p0_ref, p1_ref, w_ref, b_ref, out_ref):
    nf = p0_ref[...] + p1_ref[...]
    out_ref[...] = (jnp.dot(nf, w_ref[...], preferred_element_type=jnp.float32)
                    + b_ref[...])


_g_call = pl.pallas_call(
    _g_body,
    grid=(_GN,),
    in_specs=[pl.BlockSpec((_BN, _H), lambda i: (i, 0)),
              pl.BlockSpec((_BN, _H), lambda i: (i, 0)),
              _full((_H, _H)), _full((1, _H))],
    out_specs=pl.BlockSpec((_BN, _H), lambda i: (i, 0)),
    out_shape=jax.ShapeDtypeStruct((_N, _H), jnp.float32),
    compiler_params=_PAR,
)


def _edge_attr(pr, pc):
    rd = pr - pc
    d = jnp.sqrt(jnp.sum(rd * rd, axis=1, keepdims=True))
    return rd, d


def _am0_body(posr_ref, posc_ref, g_ref, eew1_ref, eeb1_ref, eew2_ref,
              eeb2_ref, w1t_ref, w2_ref, b2_ref, emb_ref, msg_ref):
    rd, d = _edge_attr(posr_ref[...], posc_ref[...])
    lane = lax.broadcasted_iota(jnp.int32, rd.shape, 1)
    ea = jnp.where(lane < 3, rd, jnp.where(lane == 3, d, 0.0))
    h = _silu(jnp.dot(ea, eew1_ref[...], preferred_element_type=jnp.float32)
              + eeb1_ref[...])
    emb = (jnp.dot(h, eew2_ref[...], preferred_element_type=jnp.float32)
           + eeb2_ref[...])
    emb_ref[...] = emb
    h2 = _silu(jnp.dot(emb, w1t_ref[...], preferred_element_type=jnp.float32)
               + g_ref[...])
    msg_ref[...] = (jnp.dot(h2, w2_ref[...], preferred_element_type=jnp.float32)
                    + b2_ref[...])


_am0_call = pl.pallas_call(
    _am0_body,
    grid=(_GE,),
    in_specs=[pl.BlockSpec((_BE, 8), lambda i: (i, 0)),
              pl.BlockSpec((_BE, 8), lambda i: (i, 0)),
              pl.BlockSpec((_BE, _H), lambda i: (i, 0)),
              _full((8, _H)), _full((1, _H)), _full((_H, _H)), _full((1, _H)),
              _full((_H, _H)), _full((_H, _H)), _full((1, _H))],
    out_specs=[pl.BlockSpec((_BE, _H), lambda i: (i, 0)),
               pl.BlockSpec((_BE, _H), lambda i: (i, 0))],
    out_shape=[jax.ShapeDtypeStruct((_E, _H), jnp.float32),
               jax.ShapeDtypeStruct((_E, _H), jnp.float32)],
    compiler_params=_PAR,
)


def _msg_body(emb_ref, g_ref, w1t_ref, w2_ref, b2_ref, msg_ref):
    h = _silu(jnp.dot(emb_ref[...], w1t_ref[...],
                      preferred_element_type=jnp.float32) + g_ref[...])
    msg_ref[...] = (jnp.dot(h, w2_ref[...], preferred_element_type=jnp.float32)
                    + b2_ref[...])


_msg_call = pl.pallas_call(
    _msg_body,
    grid=(_GE,),
    in_specs=[pl.BlockSpec((_BE, _H), lambda i: (i, 0)),
              pl.BlockSpec((_BE, _H), lambda i: (i, 0)),
              _full((_H, _H)), _full((_H, _H)), _full((1, _H))],
    out_specs=pl.BlockSpec((_BE, _H), lambda i: (i, 0)),
    out_shape=jax.ShapeDtypeStruct((_E, _H), jnp.float32),
    compiler_params=_PAR,
)


def _ff_body(g_ref, w2_ref, b2_ref, posr_ref, posc_ref, pf_ref, nf_ref):
    fm8 = (jnp.dot(_silu(g_ref[...]), w2_ref[...],
                   preferred_element_type=jnp.float32) + b2_ref[...])
    fm = fm8[:, 0:1]
    rd, d = _edge_attr(posr_ref[...], posc_ref[...])
    pf = fm * (rd / (d + 1e-8))
    pf_ref[...] = pf
    nf_ref[...] = -pf


_ff_call = pl.pallas_call(
    _ff_body,
    grid=(_GE,),
    in_specs=[pl.BlockSpec((_BE, _H), lambda i: (i, 0)),
              _full((_H, 8)), _full((1, 8)),
              pl.BlockSpec((_BE, 8), lambda i: (i, 0)),
              pl.BlockSpec((_BE, 8), lambda i: (i, 0))],
    out_specs=[pl.BlockSpec((_BE, 8), lambda i: (i, 0)),
               pl.BlockSpec((_BE, 8), lambda i: (i, 0))],
    out_shape=[jax.ShapeDtypeStruct((_E, 8), jnp.float32),
               jax.ShapeDtypeStruct((_E, 8), jnp.float32)],
    compiler_params=_PAR,
)


def _fin_body(q0_ref, q1_ref, out_ref):
    out_ref[...] = (q0_ref[...] + q1_ref[...])[:, :3]


_fin_call = pl.pallas_call(
    _fin_body,
    grid=(_GN,),
    in_specs=[pl.BlockSpec((_BN, 8), lambda i: (i, 0)),
              pl.BlockSpec((_BN, 8), lambda i: (i, 0))],
    out_specs=pl.BlockSpec((_BN, 3), lambda i: (i, 0)),
    out_shape=jax.ShapeDtypeStruct((_N, 3), jnp.float32),
    compiler_params=_PAR,
)


def kernel(pos, vel, masses, edge_index, ee_w1, ee_b1, ee_w2, ee_b2,
           msg_w1, msg_b1, msg_w2, msg_b2, fd_w1, fd_b1, fd_w2, fd_b2):
    f32 = jnp.float32
    row = edge_index[0]
    col = edge_index[1]
    pos8 = jnp.concatenate([pos, jnp.zeros((_N, 5), f32)], axis=1)
    eew1p = jnp.concatenate([ee_w1, jnp.zeros((4, _H), f32)], axis=0)
    fd_w2p = jnp.concatenate([fd_w2, jnp.zeros((_H, 7), f32)], axis=1)
    fd_b2p = jnp.concatenate([fd_b2, jnp.zeros((7,), f32)]).reshape(1, 8)
    w1t = msg_w1[:, :_H, :]
    w1b = msg_w1[:, _H:, :]
    zeros128 = jnp.zeros((_N // _NS, _H), f32)
    zeros8 = jnp.zeros((1250, 8), f32)

    g0 = _g0_call(vel, w1b[0], msg_b1[0].reshape(1, _H))
    posr = _gather8(pos8, row)
    posc = _gather8(pos8, col)
    gth = _gather128(g0, row)
    emb, msg = _am0_call(posr, posc, gth, eew1p, ee_b1.reshape(1, _H),
                         ee_w2, ee_b2.reshape(1, _H), w1t[0], msg_w2[0],
                         msg_b2[0].reshape(1, _H))
    p = _scatter128(msg, col, zeros128)
    for l in range(1, _L):
        g = _g_call(p[:_N], p[_N:], w1b[l], msg_b1[l].reshape(1, _H))
        gth = _gather128(g, row)
        msg = _msg_call(emb, gth, w1t[l], msg_w2[l], msg_b2[l].reshape(1, _H))
        p = _scatter128(msg, col, zeros128)
    gf = _g_call(p[:_N], p[_N:], fd_w1, fd_b1.reshape(1, _H))
    gfr = _gather128(gf, row)
    pf, nf = _ff_call(gfr, fd_w2p, fd_b2p, posr, posc)
    q = _scatter2(pf, nf, row, col, zeros8)
    return _fin_call(q[:_N], q[_N:])


# speedup vs baseline: 3.2209x; 1.1469x over previous
import functools

import jax
import jax.numpy as jnp
from jax import lax
from jax.experimental import pallas as pl
from jax.experimental.pallas import tpu as pltpu
from jax.experimental.pallas import tpu_sc as plsc

_N = 10000
_E = 320000
_H = 128
_L = 4

_NC = 2
_NS = 16
_NW = _NC * _NS
_C = 128
_NFULL = 78
_EW = _NFULL * _C
_TAIL0 = _NW * _EW
_NTAIL = (_E - _TAIL0) // _C


def _sc_mesh():
    return plsc.VectorSubcoreMesh(
        core_axis_name="c", subcore_axis_name="s",
        num_cores=_NC, num_subcores=_NS)


_SC_PARAMS = pltpu.CompilerParams(use_tc_tiling_on_sc=False)


_NB = 3
_NBS = 2


def _make_gather(width):
    @functools.partial(
        pl.kernel,
        out_type=jax.ShapeDtypeStruct((_E, width), jnp.float32),
        mesh=_sc_mesh(),
        compiler_params=_SC_PARAMS,
        scratch_types=[
            pltpu.VMEM((_EW,), jnp.int32),
            pltpu.VMEM((_C,), jnp.int32),
            [pltpu.VMEM((_C, width), jnp.float32)] * _NB,
            [pltpu.SemaphoreType.DMA] * _NB,
            [pltpu.SemaphoreType.DMA] * _NB,
        ],
    )
    def gather(tbl, idx_hbm, out, idx_all, idx_t, bufs, gsems, ssems):
        w = lax.axis_index("s") * _NC + lax.axis_index("c")
        base = pl.multiple_of(w * _EW, _EW)
        pltpu.sync_copy(idx_hbm.at[pl.ds(base, _EW)], idx_all)

        def ring(j, carry):
            descs = []
            for t in range(_NB):
                off = pl.multiple_of((j * _NB + t) * _C, _C)

                @pl.when(j > 0)
                def _drain_store(t=t):
                    pltpu.make_async_copy(
                        out.at[pl.ds(base, _C)], bufs[t], ssems[t]).wait()

                descs.append(pltpu.async_copy(
                    tbl.at[idx_all.at[pl.ds(off, _C)]], bufs[t], gsems[t]))
            for t in range(_NB):
                off = pl.multiple_of((j * _NB + t) * _C, _C)
                descs[t].wait()
                pltpu.async_copy(bufs[t], out.at[pl.ds(base + off, _C)],
                                 ssems[t])
            return carry

        lax.fori_loop(0, _NFULL // _NB, ring, 0)
        for t in range(_NB):
            pltpu.make_async_copy(
                out.at[pl.ds(base, _C)], bufs[t], ssems[t]).wait()

        @pl.when(w < _NTAIL)
        def _tail():
            tb = pl.multiple_of(_TAIL0 + w * _C, _C)
            pltpu.sync_copy(idx_hbm.at[pl.ds(tb, _C)], idx_t)
            pltpu.async_copy(tbl.at[idx_t], bufs[0], gsems[0]).wait()
            pltpu.sync_copy(bufs[0], out.at[pl.ds(tb, _C)])

    return gather


def _make_scatter(width, nzrow):
    nz = _N // nzrow

    @functools.partial(
        pl.kernel,
        out_type=jax.ShapeDtypeStruct((_NC * _N, width), jnp.float32),
        mesh=_sc_mesh(),
        compiler_params=_SC_PARAMS,
        scratch_types=[
            pltpu.VMEM_SHARED((_N, width), jnp.float32),
            pltpu.VMEM((_EW,), jnp.int32),
            pltpu.VMEM((_C,), jnp.int32),
            [pltpu.VMEM((_C, width), jnp.float32)] * _NBS,
            [pltpu.SemaphoreType.DMA] * _NBS,
            [pltpu.SemaphoreType.DMA] * _NBS,
        ],
    )
    def scatter(msg, idx_hbm, zeros_hbm, out,
                acc, idx_all, idx_t, bufs, lsems, asems):
        cid = lax.axis_index("c")
        sid = lax.axis_index("s")
        w = sid * _NC + cid

        @pl.when(sid < nz)
        def _zero():
            pltpu.sync_copy(zeros_hbm, acc.at[pl.ds(sid * nzrow, nzrow)])

        plsc.subcore_barrier()

        base = pl.multiple_of(w * _EW, _EW)
        pltpu.sync_copy(idx_hbm.at[pl.ds(base, _EW)], idx_all)

        def ring(j, carry):
            descs = []
            for t in range(_NBS):
                off = pl.multiple_of((j * _NBS + t) * _C, _C)

                @pl.when(j > 0)
                def _drain_add(t=t):
                    pltpu.make_async_copy(
                        msg.at[pl.ds(base, _C)], bufs[t], asems[t]).wait()

                descs.append(pltpu.async_copy(
                    msg.at[pl.ds(base + off, _C)], bufs[t], lsems[t]))
            for t in range(_NBS):
                off = pl.multiple_of((j * _NBS + t) * _C, _C)
                descs[t].wait()
                pltpu.async_copy(bufs[t], acc.at[idx_all.at[pl.ds(off, _C)]],
                                 asems[t], add=True)
            return carry

        lax.fori_loop(0, _NFULL // _NBS, ring, 0)
        for t in range(_NBS):
            pltpu.make_async_copy(
                msg.at[pl.ds(base, _C)], bufs[t], asems[t]).wait()

        @pl.when(w < _NTAIL)
        def _tail():
            tb = pl.multiple_of(_TAIL0 + w * _C, _C)
            pltpu.sync_copy(idx_hbm.at[pl.ds(tb, _C)], idx_t)
            pltpu.sync_copy(msg.at[pl.ds(tb, _C)], bufs[0])
            pltpu.sync_copy(bufs[0], acc.at[idx_t], add=True)

        plsc.subcore_barrier()

        @pl.when(sid < nz)
        def _writeback():
            pltpu.sync_copy(acc.at[pl.ds(sid * nzrow, nzrow)],
                            out.at[pl.ds(cid * _N + sid * nzrow, nzrow)])

    return scatter


def _make_scatter2():
    width = 8
    nzrow = 1250
    nz = _N // nzrow

    @functools.partial(
        pl.kernel,
        out_type=jax.ShapeDtypeStruct((_NC * _N, width), jnp.float32),
        mesh=_sc_mesh(),
        compiler_params=_SC_PARAMS,
        scratch_types=[
            pltpu.VMEM_SHARED((_N, width), jnp.float32),
            pltpu.VMEM((_EW,), jnp.int32),
            pltpu.VMEM((_C,), jnp.int32),
            [pltpu.VMEM((_C, width), jnp.float32)] * _NB,
            [pltpu.SemaphoreType.DMA] * _NB,
            [pltpu.SemaphoreType.DMA] * _NB,
        ],
    )
    def scatter2(pf, nf, row_hbm, col_hbm, zeros_hbm, out,
                 acc, idx_all, idx_t, bufs, lsems, asems):
        cid = lax.axis_index("c")
        sid = lax.axis_index("s")
        w = sid * _NC + cid

        @pl.when(sid < nz)
        def _zero():
            pltpu.sync_copy(zeros_hbm, acc.at[pl.ds(sid * nzrow, nzrow)])

        plsc.subcore_barrier()

        base = pl.multiple_of(w * _EW, _EW)

        def phase(vals, idx_hbm):
            pltpu.sync_copy(idx_hbm.at[pl.ds(base, _EW)], idx_all)

            def ring(j, carry):
                descs = []
                for t in range(_NB):
                    off = pl.multiple_of((j * _NB + t) * _C, _C)

                    @pl.when(j > 0)
                    def _drain_add(t=t):
                        pltpu.make_async_copy(
                            vals.at[pl.ds(base, _C)], bufs[t], asems[t]).wait()

                    descs.append(pltpu.async_copy(
                        vals.at[pl.ds(base + off, _C)], bufs[t], lsems[t]))
                for t in range(_NB):
                    off = pl.multiple_of((j * _NB + t) * _C, _C)
                    descs[t].wait()
                    pltpu.async_copy(
                        bufs[t], acc.at[idx_all.at[pl.ds(off, _C)]],
                        asems[t], add=True)
                return carry

            lax.fori_loop(0, _NFULL // _NB, ring, 0)
            for t in range(_NB):
                pltpu.make_async_copy(
                    vals.at[pl.ds(base, _C)], bufs[t], asems[t]).wait()

            @pl.when(w < _NTAIL)
            def _tail():
                tb = pl.multiple_of(_TAIL0 + w * _C, _C)
                pltpu.sync_copy(idx_hbm.at[pl.ds(tb, _C)], idx_t)
                pltpu.sync_copy(vals.at[pl.ds(tb, _C)], bufs[0])
                pltpu.sync_copy(bufs[0], acc.at[idx_t], add=True)

        phase(pf, row_hbm)
        phase(nf, col_hbm)

        plsc.subcore_barrier()

        @pl.when(sid < nz)
        def _writeback():
            pltpu.sync_copy(acc.at[pl.ds(sid * nzrow, nzrow)],
                            out.at[pl.ds(cid * _N + sid * nzrow, nzrow)])

    return scatter2


_gather8 = _make_gather(8)
_gather128 = _make_gather(_H)
_scatter128 = _make_scatter(_H, _N // _NS)
_scatter2 = _make_scatter2()


_BN = 1000
_GN = _N // _BN
_BE = 2000
_GE = _E // _BE

_PAR = pltpu.CompilerParams(dimension_semantics=("parallel",))


def _silu(x):
    return x * jax.nn.sigmoid(x)


def _full(shape):
    return pl.BlockSpec(shape, lambda i: (0, 0))


def _g0_body(vel_ref, w_ref, b_ref, out_ref):
    v = vel_ref[...]
    s = jnp.sum(w_ref[...], axis=0, keepdims=True)
    vn = jnp.sqrt(jnp.sum(v * v, axis=1, keepdims=True))
    out_ref[...] = vn * s + b_ref[...]


_g0_call = pl.pallas_call(
    _g0_body,
    grid=(_GN,),
    in_specs=[pl.BlockSpec((_BN, 3), lambda i: (i, 0)),
              _full((_H, _H)), _full((1, _H))],
    out_specs=pl.BlockSpec((_BN, _H), lambda i: (i, 0)),
    out_shape=jax.ShapeDtypeStruct((_N, _H), jnp.float32),
    compiler_params=_PAR,
)


def _g_body(p0_ref, p1_ref, w_ref, b_ref, out_ref):
    nf = p0_ref[...] + p1_ref[...]
    out_ref[...] = (jnp.dot(nf, w_ref[...], preferred_element_type=jnp.float32)
                    + b_ref[...])


_g_call = pl.pallas_call(
    _g_body,
    grid=(_GN,),
    in_specs=[pl.BlockSpec((_BN, _H), lambda i: (i, 0)),
              pl.BlockSpec((_BN, _H), lambda i: (i, 0)),
              _full((_H, _H)), _full((1, _H))],
    out_specs=pl.BlockSpec((_BN, _H), lambda i: (i, 0)),
    out_shape=jax.ShapeDtypeStruct((_N, _H), jnp.float32),
    compiler_params=_PAR,
)


def _edge_attr(pr, pc):
    rd = pr - pc
    d = jnp.sqrt(jnp.sum(rd * rd, axis=1, keepdims=True))
    return rd, d


def _am0_body(posr_ref, posc_ref, g_ref, eew1_ref, eeb1_ref, eew2_ref,
              eeb2_ref, w1t_ref, w2_ref, b2_ref, emb_ref, msg_ref):
    rd, d = _edge_attr(posr_ref[...], posc_ref[...])
    lane = lax.broadcasted_iota(jnp.int32, rd.shape, 1)
    ea = jnp.where(lane < 3, rd, jnp.where(lane == 3, d, 0.0))
    h = _silu(jnp.dot(ea, eew1_ref[...], preferred_element_type=jnp.float32)
              + eeb1_ref[...])
    emb = (jnp.dot(h, eew2_ref[...], preferred_element_type=jnp.float32)
           + eeb2_ref[...])
    emb_ref[...] = emb
    h2 = _silu(jnp.dot(emb, w1t_ref[...], preferred_element_type=jnp.float32)
               + g_ref[...])
    msg_ref[...] = (jnp.dot(h2, w2_ref[...], preferred_element_type=jnp.float32)
                    + b2_ref[...])


_am0_call = pl.pallas_call(
    _am0_body,
    grid=(_GE,),
    in_specs=[pl.BlockSpec((_BE, 8), lambda i: (i, 0)),
              pl.BlockSpec((_BE, 8), lambda i: (i, 0)),
              pl.BlockSpec((_BE, _H), lambda i: (i, 0)),
              _full((8, _H)), _full((1, _H)), _full((_H, _H)), _full((1, _H)),
              _full((_H, _H)), _full((_H, _H)), _full((1, _H))],
    out_specs=[pl.BlockSpec((_BE, _H), lambda i: (i, 0)),
               pl.BlockSpec((_BE, _H), lambda i: (i, 0))],
    out_shape=[jax.ShapeDtypeStruct((_E, _H), jnp.float32),
               jax.ShapeDtypeStruct((_E, _H), jnp.float32)],
    compiler_params=_PAR,
)


def _msg_body(emb_ref, g_ref, w1t_ref, w2_ref, b2_ref, msg_ref):
    h = _silu(jnp.dot(emb_ref[...], w1t_ref[...],
                      preferred_element_type=jnp.float32) + g_ref[...])
    msg_ref[...] = (jnp.dot(h, w2_ref[...], preferred_element_type=jnp.float32)
                    + b2_ref[...])


_msg_call = pl.pallas_call(
    _msg_body,
    grid=(_GE,),
    in_specs=[pl.BlockSpec((_BE, _H), lambda i: (i, 0)),
              pl.BlockSpec((_BE, _H), lambda i: (i, 0)),
              _full((_H, _H)), _full((_H, _H)), _full((1, _H))],
    out_specs=pl.BlockSpec((_BE, _H), lambda i: (i, 0)),
    out_shape=jax.ShapeDtypeStruct((_E, _H), jnp.float32),
    compiler_params=_PAR,
)


def _ff_body(g_ref, w2_ref, b2_ref, posr_ref, posc_ref, pf_ref, nf_ref):
    fm8 = (jnp.dot(_silu(g_ref[...]), w2_ref[...],
                   preferred_element_type=jnp.float32) + b2_ref[...])
    fm = fm8[:, 0:1]
    rd, d = _edge_attr(posr_ref[...], posc_ref[...])
    pf = fm * (rd / (d + 1e-8))
    pf_ref[...] = pf
    nf_ref[...] = -pf


_ff_call = pl.pallas_call(
    _ff_body,
    grid=(_GE,),
    in_specs=[pl.BlockSpec((_BE, _H), lambda i: (i, 0)),
              _full((_H, 8)), _full((1, 8)),
              pl.BlockSpec((_BE, 8), lambda i: (i, 0)),
              pl.BlockSpec((_BE, 8), lambda i: (i, 0))],
    out_specs=[pl.BlockSpec((_BE, 8), lambda i: (i, 0)),
               pl.BlockSpec((_BE, 8), lambda i: (i, 0))],
    out_shape=[jax.ShapeDtypeStruct((_E, 8), jnp.float32),
               jax.ShapeDtypeStruct((_E, 8), jnp.float32)],
    compiler_params=_PAR,
)


def _fin_body(q0_ref, q1_ref, out_ref):
    out_ref[...] = (q0_ref[...] + q1_ref[...])[:, :3]


_fin_call = pl.pallas_call(
    _fin_body,
    grid=(_GN,),
    in_specs=[pl.BlockSpec((_BN, 8), lambda i: (i, 0)),
              pl.BlockSpec((_BN, 8), lambda i: (i, 0))],
    out_specs=pl.BlockSpec((_BN, 3), lambda i: (i, 0)),
    out_shape=jax.ShapeDtypeStruct((_N, 3), jnp.float32),
    compiler_params=_PAR,
)


def kernel(pos, vel, masses, edge_index, ee_w1, ee_b1, ee_w2, ee_b2,
           msg_w1, msg_b1, msg_w2, msg_b2, fd_w1, fd_b1, fd_w2, fd_b2):
    f32 = jnp.float32
    row = edge_index[0]
    col = edge_index[1]
    pos8 = jnp.concatenate([pos, jnp.zeros((_N, 5), f32)], axis=1)
    eew1p = jnp.concatenate([ee_w1, jnp.zeros((4, _H), f32)], axis=0)
    fd_w2p = jnp.concatenate([fd_w2, jnp.zeros((_H, 7), f32)], axis=1)
    fd_b2p = jnp.concatenate([fd_b2, jnp.zeros((7,), f32)]).reshape(1, 8)
    w1t = msg_w1[:, :_H, :]
    w1b = msg_w1[:, _H:, :]
    zeros128 = jnp.zeros((_N // _NS, _H), f32)
    zeros8 = jnp.zeros((1250, 8), f32)

    g0 = _g0_call(vel, w1b[0], msg_b1[0].reshape(1, _H))
    posr = _gather8(pos8, row)
    posc = _gather8(pos8, col)
    gth = _gather128(g0, row)
    emb, msg = _am0_call(posr, posc, gth, eew1p, ee_b1.reshape(1, _H),
                         ee_w2, ee_b2.reshape(1, _H), w1t[0], msg_w2[0],
                         msg_b2[0].reshape(1, _H))
    p = _scatter128(msg, col, zeros128)
    for l in range(1, _L):
        g = _g_call(p[:_N], p[_N:], w1b[l], msg_b1[l].reshape(1, _H))
        gth = _gather128(g, row)
        msg = _msg_call(emb, gth, w1t[l], msg_w2[l], msg_b2[l].reshape(1, _H))
        p = _scatter128(msg, col, zeros128)
    gf = _g_call(p[:_N], p[_N:], fd_w1, fd_b1.reshape(1, _H))
    gfr = _gather128(gf, row)
    pf, nf = _ff_call(gfr, fd_w2p, fd_b2p, posr, posc)
    q = _scatter2(pf, nf, row, col, zeros8)
    return _fin_call(q[:_N], q[_N:])
